# baseline (speedup 1.0000x reference)
import functools

import jax
import jax.numpy as jnp
from jax import lax
from jax.experimental import pallas as pl
from jax.experimental.pallas import tpu as pltpu
from jax.experimental.pallas import tpu_sc as plsc

_ETA = 0.01
_EPS = 1e-12
_D_OUT = 1000
_D_PAD = 1024
_DH = 512
_NEC = 16
_NBC = 2
_BC = 128


def _sim_body(sn_ref, x_ref, out_ref):
    x = x_ref[...]
    xn_blk = x / (jnp.sqrt(jnp.sum(x * x, axis=1, keepdims=True)) + _EPS)
    s = sn_ref[...]
    sn = s / (jnp.sqrt(jnp.sum(s * s, axis=1, keepdims=True)) + _EPS)
    cos = lax.dot_general(sn, xn_blk, (((1,), (1,)), ((), ())),
                          preferred_element_type=jnp.float32)
    out_ref[...] = ((cos + 1.0) * 0.5)[None]


def _sim_call(sign_features, heatmap):
    S, F = sign_features.shape
    B = heatmap.shape[0]
    SB = 1000
    return pl.pallas_call(
        _sim_body,
        grid=(S // SB, _NBC),
        in_specs=[
            pl.BlockSpec((SB, F), lambda i, j: (i, 0)),
            pl.BlockSpec((_BC, F), lambda i, j: (j, 0)),
        ],
        out_specs=pl.BlockSpec((1, SB, _BC), lambda i, j: (j, i, 0)),
        out_shape=jax.ShapeDtypeStruct((_NBC, S, _BC), jnp.float32),
    )(sign_features, heatmap)


def _edge_call(sim4, didx3, sidx3, w2):
    info = plsc.get_sparse_core_info()
    NC, NS = info.num_cores, info.num_subcores
    NW = NC * NS
    assert NW == _NEC * _NBC
    _, NB, G = didx3.shape
    EC = NB * G

    @functools.partial(
        pl.kernel,
        out_type=jax.ShapeDtypeStruct((_NBC, _NEC, 2, _DH, _BC), jnp.float32),
        mesh=plsc.VectorSubcoreMesh(core_axis_name="c", subcore_axis_name="s"),
        scratch_types=[
            pltpu.VMEM((G,), jnp.int32),
            pltpu.VMEM((G,), jnp.int32),
            pltpu.VMEM((EC,), jnp.float32),
            pltpu.VMEM((G, _BC), jnp.float32),
            pltpu.VMEM((_DH, _BC), jnp.float32),
            pltpu.SemaphoreType.DMA,
        ],
    )
    def k(sim_hbm, didx_hbm, sidx_hbm, w_hbm, out_hbm,
          sidx_buf, didx_buf, w_v, rows_v, acc_v, sem):
        c = lax.axis_index("c")
        s = lax.axis_index("s")
        wid = s * NC + c
        ec = lax.rem(wid, _NEC)
        bc = wid // _NEC
        pltpu.sync_copy(w_hbm.at[ec], w_v)
        zero16 = jnp.zeros((16,), jnp.float32)
        dnums = lax.GatherDimensionNumbers(
            offset_dims=(), collapsed_slice_dims=(0,), start_index_map=(0,))

        def pass_body(p, carry0):
            d_lo = p * _DH

            def zrow(r, carry):
                for jj in range(_BC // 16):
                    acc_v[r, pl.ds(jj * 16, 16)] = zero16
                return carry

            lax.fori_loop(0, _DH, zrow, 0)

            def batch_body(b, carry):
                pltpu.sync_copy(sidx_hbm.at[ec].at[b], sidx_buf)
                pltpu.sync_copy(didx_hbm.at[ec].at[b], didx_buf)
                pltpu.async_copy(sim_hbm.at[bc].at[sidx_buf], rows_v,
                                 sem).wait()

                def group_body(gidx, c2):
                    d16 = didx_buf[pl.ds(gidx * 16, 16)] - d_lo
                    w16 = w_v[pl.ds(b * G + gidx * 16, 16)]
                    for l in range(16):
                        d = d16[l]

                        @pl.when(jnp.logical_and(d >= 0, d < _DH))
                        def _():
                            wspl = lax.gather(
                                w16, jnp.full((16, 1), l, jnp.int32),
                                dnums, (1,),
                                mode=lax.GatherScatterMode.PROMISE_IN_BOUNDS)
                            e = gidx * 16 + l
                            for jj in range(_BC // 16):
                                sl = pl.ds(jj * 16, 16)
                                acc_v[d, sl] = (acc_v[d, sl]
                                                + rows_v[e, sl] * wspl)

                    return c2

                lax.fori_loop(0, G // 16, group_body, 0)
                return carry

            lax.fori_loop(0, NB, batch_body, 0)
            pltpu.sync_copy(acc_v, out_hbm.at[bc].at[ec].at[p])
            return carry0

        lax.fori_loop(0, 2, pass_body, 0)

    return k(sim4, didx3, sidx3, w2)


def _fin_body(p_ref, g_ref, out_ref):
    acc = jnp.sum(p_ref[...][0, :, 0], axis=0)
    g = g_ref[...]
    gn = jnp.sqrt(jnp.sum(g * g, axis=1))
    row = lax.broadcasted_iota(jnp.int32, (_BC, _BC), 0)
    col = lax.broadcasted_iota(jnp.int32, (_BC, _BC), 1)
    m = jnp.where(row == col, (_ETA * gn)[:, None], 0.0)
    out_ref[...] = lax.dot_general(m, acc, (((1,), (1,)), ((), ())),
                                   preferred_element_type=jnp.float32)


def _fin_call(partials, grad):
    B, F = grad.shape
    return pl.pallas_call(
        _fin_body,
        grid=(_NBC, 2),
        in_specs=[
            pl.BlockSpec((1, _NEC, 1, _DH, _BC), lambda j, p: (j, 0, p, 0, 0)),
            pl.BlockSpec((_BC, F), lambda j, p: (j, 0)),
        ],
        out_specs=pl.BlockSpec((_BC, _DH), lambda j, p: (j, p)),
        out_shape=jax.ShapeDtypeStruct((B, _D_PAD), jnp.float32),
    )(partials, grad)


def kernel(heatmap_features_batch, grad_output_batch, sign_features,
           disease_idx, sign_idx, edge_weight, num_diseases):
    B, F = heatmap_features_batch.shape
    S = sign_features.shape[0]
    E = disease_idx.shape[0]
    G = 128
    NB = -(-E // (_NEC * G))
    E_pad = _NEC * NB * G
    pad = E_pad - E

    sim4 = _sim_call(sign_features, heatmap_features_batch)

    didx_p = jnp.concatenate([disease_idx, jnp.zeros((pad,), jnp.int32)])
    sidx_p = jnp.concatenate([sign_idx, jnp.zeros((pad,), jnp.int32)])
    w_p = jnp.concatenate([edge_weight, jnp.zeros((pad,), jnp.float32)])
    didx3 = didx_p.reshape(_NEC, NB, G)
    sidx3 = sidx_p.reshape(_NEC, NB, G)
    w2 = w_p.reshape(_NEC, NB * G)
    partials = _edge_call(sim4, didx3, sidx3, w2)

    return _fin_call(partials, grad_output_batch)[:, :_D_OUT]


# speedup vs baseline: 1.5263x; 1.5263x over previous
import functools

import jax
import jax.numpy as jnp
from jax import lax
from jax.experimental import pallas as pl
from jax.experimental.pallas import tpu as pltpu
from jax.experimental.pallas import tpu_sc as plsc

_ETA = 0.01
_EPS = 1e-12
_D_OUT = 1000
_D_PAD = 1024
_DH = 512
_NEC = 16
_NBC = 2
_BC = 128


def _sim_body(sn_ref, x_ref, out_ref):
    x = x_ref[...]
    xn_blk = x / (jnp.sqrt(jnp.sum(x * x, axis=1, keepdims=True)) + _EPS)
    s = sn_ref[...]
    sn = s / (jnp.sqrt(jnp.sum(s * s, axis=1, keepdims=True)) + _EPS)
    cos = lax.dot_general(sn, xn_blk, (((1,), (1,)), ((), ())),
                          preferred_element_type=jnp.float32)
    out_ref[...] = ((cos + 1.0) * 0.5)[None]


def _sim_call(sign_features, heatmap):
    S, F = sign_features.shape
    B = heatmap.shape[0]
    SB = 1000
    return pl.pallas_call(
        _sim_body,
        grid=(S // SB, _NBC),
        in_specs=[
            pl.BlockSpec((SB, F), lambda i, j: (i, 0)),
            pl.BlockSpec((_BC, F), lambda i, j: (j, 0)),
        ],
        out_specs=pl.BlockSpec((1, SB, _BC), lambda i, j: (j, i, 0)),
        out_shape=jax.ShapeDtypeStruct((_NBC, S, _BC), jnp.float32),
    )(sign_features, heatmap)


def _edge_call(sim4, didx3, sidx3, w2):
    info = plsc.get_sparse_core_info()
    NC, NS = info.num_cores, info.num_subcores
    NW = NC * NS
    assert NW == _NEC * _NBC
    _, NB, G = didx3.shape
    EC = NB * G

    @functools.partial(
        pl.kernel,
        out_type=jax.ShapeDtypeStruct((_NBC, _NEC, 2, _DH, _BC), jnp.float32),
        mesh=plsc.VectorSubcoreMesh(core_axis_name="c", subcore_axis_name="s"),
        scratch_types=[
            pltpu.VMEM((NB, G), jnp.int32),
            pltpu.VMEM((NB, G), jnp.int32),
            pltpu.VMEM((EC,), jnp.float32),
            pltpu.VMEM((G, _BC), jnp.float32),
            pltpu.VMEM((G, _BC), jnp.float32),
            pltpu.VMEM((_DH, _BC), jnp.float32),
            pltpu.SemaphoreType.DMA,
            pltpu.SemaphoreType.DMA,
        ],
    )
    def k(sim_hbm, didx_hbm, sidx_hbm, w_hbm, out_hbm,
          sidx_v, didx_v, w_v, rows0, rows1, acc_v, sem0, sem1):
        c = lax.axis_index("c")
        s = lax.axis_index("s")
        wid = s * NC + c
        ec = lax.rem(wid, _NEC)
        bc = wid // _NEC
        pltpu.sync_copy(sidx_hbm.at[ec], sidx_v)
        pltpu.sync_copy(didx_hbm.at[ec], didx_v)
        pltpu.sync_copy(w_hbm.at[ec], w_v)
        zero16 = jnp.zeros((16,), jnp.float32)
        dnums = lax.GatherDimensionNumbers(
            offset_dims=(), collapsed_slice_dims=(0,), start_index_map=(0,))

        def gather_rows(b, rows, sem):
            return pltpu.async_copy(sim_hbm.at[bc].at[sidx_v.at[b]], rows,
                                    sem)

        def pass_body(p, carry0):
            d_lo = p * _DH

            def zrow(r, carry):
                for jj in range(_BC // 16):
                    acc_v[r, pl.ds(jj * 16, 16)] = zero16
                return carry

            lax.fori_loop(0, _DH, zrow, 0)
            gather_rows(0, rows0, sem0)

            def pair_body(h, carry):
                b0 = 2 * h
                gather_rows(b0 + 1, rows1, sem1)
                pltpu.make_async_copy(
                    sim_hbm.at[bc].at[sidx_v.at[b0]], rows0, sem0).wait()
                process_pass(b0, rows0, d_lo)

                @pl.when(h < NB // 2 - 1)
                def _():
                    gather_rows(b0 + 2, rows0, sem0)

                pltpu.make_async_copy(
                    sim_hbm.at[bc].at[sidx_v.at[b0 + 1]], rows1, sem1).wait()
                process_pass(b0 + 1, rows1, d_lo)
                return carry

            lax.fori_loop(0, NB // 2, pair_body, 0)
            pltpu.sync_copy(acc_v, out_hbm.at[bc].at[ec].at[p])
            return carry0

        def process_pass(b, rows, d_lo):
            def group_body(gidx, c2):
                d16 = didx_v[b, pl.ds(gidx * 16, 16)] - d_lo
                w16 = w_v[pl.ds(b * G + gidx * 16, 16)]
                for l in range(16):
                    d = d16[l]

                    @pl.when(jnp.logical_and(d >= 0, d < _DH))
                    def _():
                        wspl = lax.gather(
                            w16, jnp.full((16, 1), l, jnp.int32),
                            dnums, (1,),
                            mode=lax.GatherScatterMode.PROMISE_IN_BOUNDS)
                        e = gidx * 16 + l
                        for jj in range(_BC // 16):
                            sl = pl.ds(jj * 16, 16)
                            plsc.addupdate(acc_v.at[d, sl],
                                           rows[e, sl] * wspl)

                return c2

            lax.fori_loop(0, G // 16, group_body, 0)

        lax.fori_loop(0, 2, pass_body, 0)

    return k(sim4, didx3, sidx3, w2)


def _fin_body(p_ref, g_ref, out_ref):
    acc = jnp.sum(p_ref[...][0, :, 0], axis=0)
    g = g_ref[...]
    gn = jnp.sqrt(jnp.sum(g * g, axis=1))
    row = lax.broadcasted_iota(jnp.int32, (_BC, _BC), 0)
    col = lax.broadcasted_iota(jnp.int32, (_BC, _BC), 1)
    m = jnp.where(row == col, (_ETA * gn)[:, None], 0.0)
    out_ref[...] = lax.dot_general(m, acc, (((1,), (1,)), ((), ())),
                                   preferred_element_type=jnp.float32)


def _fin_call(partials, grad):
    B, F = grad.shape
    return pl.pallas_call(
        _fin_body,
        grid=(_NBC, 2),
        in_specs=[
            pl.BlockSpec((1, _NEC, 1, _DH, _BC), lambda j, p: (j, 0, p, 0, 0)),
            pl.BlockSpec((_BC, F), lambda j, p: (j, 0)),
        ],
        out_specs=pl.BlockSpec((_BC, _DH), lambda j, p: (j, p)),
        out_shape=jax.ShapeDtypeStruct((B, _D_PAD), jnp.float32),
    )(partials, grad)


def kernel(heatmap_features_batch, grad_output_batch, sign_features,
           disease_idx, sign_idx, edge_weight, num_diseases):
    B, F = heatmap_features_batch.shape
    S = sign_features.shape[0]
    E = disease_idx.shape[0]
    G = 128
    NB = -(-E // (_NEC * G))
    NB += NB % 2
    E_pad = _NEC * NB * G
    pad = E_pad - E

    sim4 = _sim_call(sign_features, heatmap_features_batch)

    didx_p = jnp.concatenate([disease_idx, jnp.zeros((pad,), jnp.int32)])
    sidx_p = jnp.concatenate([sign_idx, jnp.zeros((pad,), jnp.int32)])
    w_p = jnp.concatenate([edge_weight, jnp.zeros((pad,), jnp.float32)])
    didx3 = didx_p.reshape(_NEC, NB, G)
    sidx3 = sidx_p.reshape(_NEC, NB, G)
    w2 = w_p.reshape(_NEC, NB * G)
    partials = _edge_call(sim4, didx3, sidx3, w2)

    return _fin_call(partials, grad_output_batch)[:, :_D_OUT]


# speedup vs baseline: 2.2460x; 1.4715x over previous
import functools

import jax
import jax.numpy as jnp
from jax import lax
from jax.experimental import pallas as pl
from jax.experimental.pallas import tpu as pltpu
from jax.experimental.pallas import tpu_sc as plsc

_ETA = 0.01
_EPS = 1e-12
_D_OUT = 1000
_D_PAD = 1024
_DH = 512
_NEC = 16
_NBC = 2
_BC = 128


def _sim_body(sn_ref, x_ref, out_ref):
    x = x_ref[...]
    xn_blk = x / (jnp.sqrt(jnp.sum(x * x, axis=1, keepdims=True)) + _EPS)
    s = sn_ref[...]
    sn = s / (jnp.sqrt(jnp.sum(s * s, axis=1, keepdims=True)) + _EPS)
    cos = lax.dot_general(sn, xn_blk, (((1,), (1,)), ((), ())),
                          preferred_element_type=jnp.float32)
    out_ref[...] = ((cos + 1.0) * 0.5)[None]


def _sim_call(sign_features, heatmap):
    S, F = sign_features.shape
    B = heatmap.shape[0]
    SB = 1000
    return pl.pallas_call(
        _sim_body,
        grid=(S // SB, _NBC),
        in_specs=[
            pl.BlockSpec((SB, F), lambda i, j: (i, 0)),
            pl.BlockSpec((_BC, F), lambda i, j: (j, 0)),
        ],
        out_specs=pl.BlockSpec((1, SB, _BC), lambda i, j: (j, i, 0)),
        out_shape=jax.ShapeDtypeStruct((_NBC, S, _BC), jnp.float32),
    )(sign_features, heatmap)


def _edge_call(sim4, didx3, sidx3, w2):
    info = plsc.get_sparse_core_info()
    NC, NS = info.num_cores, info.num_subcores
    NW = NC * NS
    assert NW == _NEC * _NBC
    _, NB, G = didx3.shape
    EC = NB * G

    @functools.partial(
        pl.kernel,
        out_type=jax.ShapeDtypeStruct((_NBC, _NEC, 2, _DH, _BC), jnp.float32),
        mesh=plsc.VectorSubcoreMesh(core_axis_name="c", subcore_axis_name="s"),
        scratch_types=[
            pltpu.VMEM((NB, G), jnp.int32),
            pltpu.VMEM((NB, G), jnp.int32),
            pltpu.VMEM((EC,), jnp.float32),
            pltpu.VMEM((G, _BC), jnp.float32),
            pltpu.VMEM((G, _BC), jnp.float32),
            pltpu.VMEM((_DH, _BC), jnp.float32),
            pltpu.SemaphoreType.DMA,
            pltpu.SemaphoreType.DMA,
        ],
    )
    def k(sim_hbm, didx_hbm, sidx_hbm, w_hbm, out_hbm,
          sidx_v, didx_v, w_v, rows0, rows1, acc_v, sem0, sem1):
        c = lax.axis_index("c")
        s = lax.axis_index("s")
        wid = s * NC + c
        ec = lax.rem(wid, _NEC)
        bc = wid // _NEC
        pltpu.sync_copy(sidx_hbm.at[ec], sidx_v)
        pltpu.sync_copy(didx_hbm.at[ec], didx_v)
        pltpu.sync_copy(w_hbm.at[ec], w_v)
        zero16 = jnp.zeros((16,), jnp.float32)
        dnums = lax.GatherDimensionNumbers(
            offset_dims=(), collapsed_slice_dims=(0,), start_index_map=(0,))

        def gather_rows(b, rows, sem):
            return pltpu.async_copy(sim_hbm.at[bc].at[sidx_v.at[b]], rows,
                                    sem)

        def pass_body(p, carry0):
            d_lo = p * _DH

            def zrow(r, carry):
                for jj in range(_BC // 16):
                    acc_v[r, pl.ds(jj * 16, 16)] = zero16
                return carry

            lax.fori_loop(0, _DH, zrow, 0)
            gather_rows(0, rows0, sem0)

            def pair_body(h, carry):
                b0 = 2 * h
                gather_rows(b0 + 1, rows1, sem1)
                pltpu.make_async_copy(
                    sim_hbm.at[bc].at[sidx_v.at[b0]], rows0, sem0).wait()
                process_pass(b0, rows0, d_lo)

                @pl.when(h < NB // 2 - 1)
                def _():
                    gather_rows(b0 + 2, rows0, sem0)

                pltpu.make_async_copy(
                    sim_hbm.at[bc].at[sidx_v.at[b0 + 1]], rows1, sem1).wait()
                process_pass(b0 + 1, rows1, d_lo)
                return carry

            lax.fori_loop(0, NB // 2, pair_body, 0)
            pltpu.sync_copy(acc_v, out_hbm.at[bc].at[ec].at[p])
            return carry0

        def process_pass(b, rows, d_lo):
            def group_body(gidx, c2):
                d16 = didx_v[b, pl.ds(gidx * 16, 16)] - d_lo
                w16 = w_v[pl.ds(b * G + gidx * 16, 16)]
                dl = [d16[l] for l in range(16)]
                wspl = [
                    lax.gather(w16, jnp.full((16, 1), l, jnp.int32),
                               dnums, (1,),
                               mode=lax.GatherScatterMode.PROMISE_IN_BOUNDS)
                    for l in range(16)
                ]
                for l in range(16):
                    d = dl[l]

                    @pl.when(jnp.logical_and(d >= 0, d < _DH))
                    def _():
                        e = gidx * 16 + l
                        prods = [rows[e, pl.ds(jj * 16, 16)] * wspl[l]
                                 for jj in range(_BC // 16)]
                        for jj in range(_BC // 16):
                            plsc.addupdate(acc_v.at[d, pl.ds(jj * 16, 16)],
                                           prods[jj])

                return c2

            lax.fori_loop(0, G // 16, group_body, 0)

        lax.fori_loop(0, 2, pass_body, 0)

    return k(sim4, didx3, sidx3, w2)


def _fin_body(p_ref, g_ref, out_ref):
    acc = jnp.sum(p_ref[...][0, :, 0], axis=0)
    g = g_ref[...]
    gn = jnp.sqrt(jnp.sum(g * g, axis=1))
    row = lax.broadcasted_iota(jnp.int32, (_BC, _BC), 0)
    col = lax.broadcasted_iota(jnp.int32, (_BC, _BC), 1)
    m = jnp.where(row == col, (_ETA * gn)[:, None], 0.0)
    out_ref[...] = lax.dot_general(m, acc, (((1,), (1,)), ((), ())),
                                   preferred_element_type=jnp.float32)


def _fin_call(partials, grad):
    B, F = grad.shape
    return pl.pallas_call(
        _fin_body,
        grid=(_NBC, 2),
        in_specs=[
            pl.BlockSpec((1, _NEC, 1, _DH, _BC), lambda j, p: (j, 0, p, 0, 0)),
            pl.BlockSpec((_BC, F), lambda j, p: (j, 0)),
        ],
        out_specs=pl.BlockSpec((_BC, _DH), lambda j, p: (j, p)),
        out_shape=jax.ShapeDtypeStruct((B, _D_PAD), jnp.float32),
    )(partials, grad)


def kernel(heatmap_features_batch, grad_output_batch, sign_features,
           disease_idx, sign_idx, edge_weight, num_diseases):
    B, F = heatmap_features_batch.shape
    S = sign_features.shape[0]
    E = disease_idx.shape[0]
    G = 128
    NB = -(-E // (_NEC * G))
    NB += NB % 2
    E_pad = _NEC * NB * G
    pad = E_pad - E

    sim4 = _sim_call(sign_features, heatmap_features_batch)

    didx_p = jnp.concatenate([disease_idx, jnp.zeros((pad,), jnp.int32)])
    sidx_p = jnp.concatenate([sign_idx, jnp.zeros((pad,), jnp.int32)])
    w_p = jnp.concatenate([edge_weight, jnp.zeros((pad,), jnp.float32)])
    didx3 = didx_p.reshape(_NEC, NB, G)
    sidx3 = sidx_p.reshape(_NEC, NB, G)
    w2 = w_p.reshape(_NEC, NB * G)
    partials = _edge_call(sim4, didx3, sidx3, w2)

    return _fin_call(partials, grad_output_batch)[:, :_D_OUT]


# speedup vs baseline: 2.2488x; 1.0012x over previous
import functools

import jax
import jax.numpy as jnp
from jax import lax
from jax.experimental import pallas as pl
from jax.experimental.pallas import tpu as pltpu
from jax.experimental.pallas import tpu_sc as plsc

_ETA = 0.01
_EPS = 1e-12
_D_OUT = 1000
_D_PAD = 1024
_DH = 512
_NEC = 16
_NBC = 2
_BC = 128


def _sim_body(sn_ref, x_ref, out_ref):
    x = x_ref[...]
    xn_blk = x / (jnp.sqrt(jnp.sum(x * x, axis=1, keepdims=True)) + _EPS)
    s = sn_ref[...]
    sn = s / (jnp.sqrt(jnp.sum(s * s, axis=1, keepdims=True)) + _EPS)
    cos = lax.dot_general(sn, xn_blk, (((1,), (1,)), ((), ())),
                          preferred_element_type=jnp.float32)
    out_ref[...] = ((cos + 1.0) * 0.5)[None]


def _sim_call(sign_features, heatmap):
    S, F = sign_features.shape
    B = heatmap.shape[0]
    SB = 1000
    return pl.pallas_call(
        _sim_body,
        grid=(S // SB, _NBC),
        in_specs=[
            pl.BlockSpec((SB, F), lambda i, j: (i, 0)),
            pl.BlockSpec((_BC, F), lambda i, j: (j, 0)),
        ],
        out_specs=pl.BlockSpec((1, SB, _BC), lambda i, j: (j, i, 0)),
        out_shape=jax.ShapeDtypeStruct((_NBC, S, _BC), jnp.float32),
    )(sign_features, heatmap)


def _edge_call(sim4, didx3, sidx3, w2):
    info = plsc.get_sparse_core_info()
    NC, NS = info.num_cores, info.num_subcores
    NW = NC * NS
    assert NW == _NEC * _NBC
    _, NB, G = didx3.shape
    EC = NB * G

    @functools.partial(
        pl.kernel,
        out_type=jax.ShapeDtypeStruct((_NBC, _NEC, 2, _DH, _BC), jnp.float32),
        mesh=plsc.VectorSubcoreMesh(core_axis_name="c", subcore_axis_name="s"),
        scratch_types=[
            pltpu.VMEM((NB, G), jnp.int32),
            pltpu.VMEM((NB, G), jnp.int32),
            pltpu.VMEM((EC,), jnp.float32),
            pltpu.VMEM((G, _BC), jnp.float32),
            pltpu.VMEM((G, _BC), jnp.float32),
            pltpu.VMEM((_DH + 1, _BC), jnp.float32),
            pltpu.SemaphoreType.DMA,
            pltpu.SemaphoreType.DMA,
        ],
    )
    def k(sim_hbm, didx_hbm, sidx_hbm, w_hbm, out_hbm,
          sidx_v, didx_v, w_v, rows0, rows1, acc_v, sem0, sem1):
        c = lax.axis_index("c")
        s = lax.axis_index("s")
        wid = s * NC + c
        ec = lax.rem(wid, _NEC)
        bc = wid // _NEC
        pltpu.sync_copy(sidx_hbm.at[ec], sidx_v)
        pltpu.sync_copy(didx_hbm.at[ec], didx_v)
        pltpu.sync_copy(w_hbm.at[ec], w_v)
        zero16 = jnp.zeros((16,), jnp.float32)
        dnums = lax.GatherDimensionNumbers(
            offset_dims=(), collapsed_slice_dims=(0,), start_index_map=(0,))

        def gather_rows(b, rows, sem):
            return pltpu.async_copy(sim_hbm.at[bc].at[sidx_v.at[b]], rows,
                                    sem)

        def pass_body(p, carry0):
            d_lo = p * _DH

            def zrow(r, carry):
                for jj in range(_BC // 16):
                    acc_v[r, pl.ds(jj * 16, 16)] = zero16
                return carry

            lax.fori_loop(0, _DH + 1, zrow, 0)
            gather_rows(0, rows0, sem0)

            def pair_body(h, carry):
                b0 = 2 * h
                gather_rows(b0 + 1, rows1, sem1)
                pltpu.make_async_copy(
                    sim_hbm.at[bc].at[sidx_v.at[b0]], rows0, sem0).wait()
                process_pass(b0, rows0, d_lo)

                @pl.when(h < NB // 2 - 1)
                def _():
                    gather_rows(b0 + 2, rows0, sem0)

                pltpu.make_async_copy(
                    sim_hbm.at[bc].at[sidx_v.at[b0 + 1]], rows1, sem1).wait()
                process_pass(b0 + 1, rows1, d_lo)
                return carry

            lax.fori_loop(0, NB // 2, pair_body, 0)
            pltpu.sync_copy(acc_v.at[pl.ds(0, _DH)],
                            out_hbm.at[bc].at[ec].at[p])
            return carry0

        def process_pass(b, rows, d_lo):
            nch = _BC // 16

            def group_body(gidx, c2):
                d16 = didx_v[b, pl.ds(gidx * 16, 16)] - d_lo
                w16 = w_v[pl.ds(b * G + gidx * 16, 16)]
                dl = [d16[l] for l in range(16)]
                dd = [jnp.where(jnp.logical_and(dl[l] >= 0, dl[l] < _DH),
                                dl[l], _DH) for l in range(16)]
                wspl = [
                    lax.gather(w16, jnp.full((16, 1), l, jnp.int32),
                               dnums, (1,),
                               mode=lax.GatherScatterMode.PROMISE_IN_BOUNDS)
                    for l in range(16)
                ]

                def lane_prods(l):
                    e = gidx * 16 + l
                    return [rows[e, pl.ds(jj * 16, 16)] * wspl[l]
                            for jj in range(nch)]

                prods = lane_prods(0)
                for l in range(16):
                    nxt = lane_prods(l + 1) if l < 15 else None
                    for jj in range(nch):
                        plsc.addupdate(acc_v.at[dd[l], pl.ds(jj * 16, 16)],
                                       prods[jj])
                    prods = nxt
                return c2

            lax.fori_loop(0, G // 16, group_body, 0)

        lax.fori_loop(0, 2, pass_body, 0)

    return k(sim4, didx3, sidx3, w2)


def _fin_body(p_ref, g_ref, out_ref):
    acc = jnp.sum(p_ref[...][0, :, 0], axis=0)
    g = g_ref[...]
    gn = jnp.sqrt(jnp.sum(g * g, axis=1))
    row = lax.broadcasted_iota(jnp.int32, (_BC, _BC), 0)
    col = lax.broadcasted_iota(jnp.int32, (_BC, _BC), 1)
    m = jnp.where(row == col, (_ETA * gn)[:, None], 0.0)
    out_ref[...] = lax.dot_general(m, acc, (((1,), (1,)), ((), ())),
                                   preferred_element_type=jnp.float32)


def _fin_call(partials, grad):
    B, F = grad.shape
    return pl.pallas_call(
        _fin_body,
        grid=(_NBC, 2),
        in_specs=[
            pl.BlockSpec((1, _NEC, 1, _DH, _BC), lambda j, p: (j, 0, p, 0, 0)),
            pl.BlockSpec((_BC, F), lambda j, p: (j, 0)),
        ],
        out_specs=pl.BlockSpec((_BC, _DH), lambda j, p: (j, p)),
        out_shape=jax.ShapeDtypeStruct((B, _D_PAD), jnp.float32),
    )(partials, grad)


def kernel(heatmap_features_batch, grad_output_batch, sign_features,
           disease_idx, sign_idx, edge_weight, num_diseases):
    B, F = heatmap_features_batch.shape
    S = sign_features.shape[0]
    E = disease_idx.shape[0]
    G = 128
    NB = -(-E // (_NEC * G))
    NB += NB % 2
    E_pad = _NEC * NB * G
    pad = E_pad - E

    sim4 = _sim_call(sign_features, heatmap_features_batch)

    didx_p = jnp.concatenate([disease_idx, jnp.zeros((pad,), jnp.int32)])
    sidx_p = jnp.concatenate([sign_idx, jnp.zeros((pad,), jnp.int32)])
    w_p = jnp.concatenate([edge_weight, jnp.zeros((pad,), jnp.float32)])
    didx3 = didx_p.reshape(_NEC, NB, G)
    sidx3 = sidx_p.reshape(_NEC, NB, G)
    w2 = w_p.reshape(_NEC, NB * G)
    partials = _edge_call(sim4, didx3, sidx3, w2)

    return _fin_call(partials, grad_output_batch)[:, :_D_OUT]


# speedup vs baseline: 2.2501x; 1.0006x over previous
import functools

import jax
import jax.numpy as jnp
from jax import lax
from jax.experimental import pallas as pl
from jax.experimental.pallas import tpu as pltpu
from jax.experimental.pallas import tpu_sc as plsc

_ETA = 0.01
_EPS = 1e-12
_D_OUT = 1000
_D_PAD = 1024
_DH = 512
_NEC = 16
_NBC = 2
_BC = 128


def _sim_body(sn_ref, x_ref, out_ref):
    x = x_ref[...]
    xn_blk = x / (jnp.sqrt(jnp.sum(x * x, axis=1, keepdims=True)) + _EPS)
    s = sn_ref[...]
    sn = s / (jnp.sqrt(jnp.sum(s * s, axis=1, keepdims=True)) + _EPS)
    cos = lax.dot_general(sn, xn_blk, (((1,), (1,)), ((), ())),
                          preferred_element_type=jnp.float32)
    out_ref[...] = ((cos + 1.0) * 0.5)[None]


def _sim_call(sign_features, heatmap):
    S, F = sign_features.shape
    B = heatmap.shape[0]
    SB = 1000
    return pl.pallas_call(
        _sim_body,
        grid=(S // SB, _NBC),
        in_specs=[
            pl.BlockSpec((SB, F), lambda i, j: (i, 0)),
            pl.BlockSpec((_BC, F), lambda i, j: (j, 0)),
        ],
        out_specs=pl.BlockSpec((1, SB, _BC), lambda i, j: (j, i, 0)),
        out_shape=jax.ShapeDtypeStruct((_NBC, S, _BC), jnp.float32),
    )(sign_features, heatmap)


def _edge_call(sim4, didx3, sidx3, w2):
    info = plsc.get_sparse_core_info()
    NC, NS = info.num_cores, info.num_subcores
    NW = NC * NS
    assert NW == _NEC * _NBC
    _, NB, G = didx3.shape
    EC = NB * G

    @functools.partial(
        pl.kernel,
        out_type=jax.ShapeDtypeStruct((_NBC, _NEC, 2, _DH, _BC), jnp.float32),
        mesh=plsc.VectorSubcoreMesh(core_axis_name="c", subcore_axis_name="s"),
        compiler_params=pltpu.CompilerParams(needs_layout_passes=False),
        scratch_types=[
            pltpu.VMEM((NB, G), jnp.int32),
            pltpu.VMEM((NB, G), jnp.int32),
            pltpu.VMEM((EC,), jnp.float32),
            pltpu.VMEM((G, _BC), jnp.float32),
            pltpu.VMEM((G, _BC), jnp.float32),
            pltpu.VMEM((_DH + 1, _BC), jnp.float32),
            pltpu.SemaphoreType.DMA,
            pltpu.SemaphoreType.DMA,
        ],
    )
    def k(sim_hbm, didx_hbm, sidx_hbm, w_hbm, out_hbm,
          sidx_v, didx_v, w_v, rows0, rows1, acc_v, sem0, sem1):
        c = lax.axis_index("c")
        s = lax.axis_index("s")
        wid = s * NC + c
        ec = lax.rem(wid, _NEC)
        bc = wid // _NEC
        pltpu.sync_copy(sidx_hbm.at[ec], sidx_v)
        pltpu.sync_copy(didx_hbm.at[ec], didx_v)
        pltpu.sync_copy(w_hbm.at[ec], w_v)
        zero16 = jnp.zeros((16,), jnp.float32)
        dnums = lax.GatherDimensionNumbers(
            offset_dims=(), collapsed_slice_dims=(0,), start_index_map=(0,))

        def gather_rows(b, rows, sem):
            return pltpu.async_copy(sim_hbm.at[bc].at[sidx_v.at[b]], rows,
                                    sem)

        def pass_body(p, carry0):
            d_lo = p * _DH

            def zrow(r, carry):
                for jj in range(_BC // 16):
                    acc_v[r, pl.ds(jj * 16, 16)] = zero16
                return carry

            lax.fori_loop(0, _DH + 1, zrow, 0)
            gather_rows(0, rows0, sem0)

            def pair_body(h, carry):
                b0 = 2 * h
                gather_rows(b0 + 1, rows1, sem1)
                pltpu.make_async_copy(
                    sim_hbm.at[bc].at[sidx_v.at[b0]], rows0, sem0).wait()
                process_pass(b0, rows0, d_lo)

                @pl.when(h < NB // 2 - 1)
                def _():
                    gather_rows(b0 + 2, rows0, sem0)

                pltpu.make_async_copy(
                    sim_hbm.at[bc].at[sidx_v.at[b0 + 1]], rows1, sem1).wait()
                process_pass(b0 + 1, rows1, d_lo)
                return carry

            lax.fori_loop(0, NB // 2, pair_body, 0)
            pltpu.sync_copy(acc_v.at[pl.ds(0, _DH)],
                            out_hbm.at[bc].at[ec].at[p])
            return carry0

        def process_pass(b, rows, d_lo):
            nch = _BC // 16

            def group_body(gidx, c2):
                d16 = didx_v[b, pl.ds(gidx * 16, 16)] - d_lo
                w16 = w_v[pl.ds(b * G + gidx * 16, 16)]
                dl = [d16[l] for l in range(16)]
                dd = [jnp.where(jnp.logical_and(dl[l] >= 0, dl[l] < _DH),
                                dl[l], _DH) for l in range(16)]
                wspl = [
                    lax.gather(w16, jnp.full((16, 1), l, jnp.int32),
                               dnums, (1,),
                               mode=lax.GatherScatterMode.PROMISE_IN_BOUNDS)
                    for l in range(16)
                ]

                def lane_prods(l):
                    e = gidx * 16 + l
                    return [rows[e, pl.ds(jj * 16, 16)] * wspl[l]
                            for jj in range(nch)]

                prods = lane_prods(0)
                for l in range(16):
                    nxt = lane_prods(l + 1) if l < 15 else None
                    for jj in range(nch):
                        plsc.addupdate(acc_v.at[dd[l], pl.ds(jj * 16, 16)],
                                       prods[jj])
                    prods = nxt
                return c2

            lax.fori_loop(0, G // 16, group_body, 0)

        lax.fori_loop(0, 2, pass_body, 0)

    return k(sim4, didx3, sidx3, w2)


def _fin_body(p_ref, g_ref, out_ref):
    acc = jnp.sum(p_ref[...][0, :, 0], axis=0)
    g = g_ref[...]
    gn = jnp.sqrt(jnp.sum(g * g, axis=1))
    row = lax.broadcasted_iota(jnp.int32, (_BC, _BC), 0)
    col = lax.broadcasted_iota(jnp.int32, (_BC, _BC), 1)
    m = jnp.where(col == row, (_ETA * gn)[:, None], 0.0)
    out_ref[...] = lax.dot_general(m, acc, (((1,), (1,)), ((), ())),
                                   preferred_element_type=jnp.float32)


def _fin_call(partials, grad):
    B, F = grad.shape
    return pl.pallas_call(
        _fin_body,
        grid=(_NBC, 2),
        in_specs=[
            pl.BlockSpec((1, _NEC, 1, _DH, _BC), lambda j, p: (j, 0, p, 0, 0)),
            pl.BlockSpec((_BC, F), lambda j, p: (j, 0)),
        ],
        out_specs=pl.BlockSpec((_BC, _DH), lambda j, p: (j, p)),
        out_shape=jax.ShapeDtypeStruct((B, _D_PAD), jnp.float32),
    )(partials, grad)


def kernel(heatmap_features_batch, grad_output_batch, sign_features,
           disease_idx, sign_idx, edge_weight, num_diseases):
    B, F = heatmap_features_batch.shape
    S = sign_features.shape[0]
    E = disease_idx.shape[0]
    G = 128
    NB = -(-E // (_NEC * G))
    NB += NB % 2
    E_pad = _NEC * NB * G
    pad = E_pad - E

    sim4 = _sim_call(sign_features, heatmap_features_batch)

    didx_p = jnp.concatenate([disease_idx, jnp.zeros((pad,), jnp.int32)])
    sidx_p = jnp.concatenate([sign_idx, jnp.zeros((pad,), jnp.int32)])
    w_p = jnp.concatenate([edge_weight, jnp.zeros((pad,), jnp.float32)])
    didx3 = didx_p.reshape(_NEC, NB, G)
    sidx3 = sidx_p.reshape(_NEC, NB, G)
    w2 = w_p.reshape(_NEC, NB * G)
    partials = _edge_call(sim4, didx3, sidx3, w2)

    return _fin_call(partials, grad_output_batch)[:, :_D_OUT]


# speedup vs baseline: 2.2529x; 1.0012x over previous
import functools

import jax
import jax.numpy as jnp
from jax import lax
from jax.experimental import pallas as pl
from jax.experimental.pallas import tpu as pltpu
from jax.experimental.pallas import tpu_sc as plsc

_ETA = 0.01
_EPS = 1e-12
_D_OUT = 1000
_D_PAD = 1024
_DH = 512
_NEC = 16
_NBC = 2
_BC = 128


def _sim_body(sn_ref, x_ref, out_ref):
    x = x_ref[...]
    xn_blk = x / (jnp.sqrt(jnp.sum(x * x, axis=1, keepdims=True)) + _EPS)
    s = sn_ref[...]
    sn = s / (jnp.sqrt(jnp.sum(s * s, axis=1, keepdims=True)) + _EPS)
    cos = lax.dot_general(sn, xn_blk, (((1,), (1,)), ((), ())),
                          preferred_element_type=jnp.float32)
    out_ref[...] = ((cos + 1.0) * 0.5)[None]


def _sim_call(sign_features, heatmap):
    S, F = sign_features.shape
    B = heatmap.shape[0]
    SB = 1000
    return pl.pallas_call(
        _sim_body,
        grid=(S // SB, _NBC),
        in_specs=[
            pl.BlockSpec((SB, F), lambda i, j: (i, 0)),
            pl.BlockSpec((_BC, F), lambda i, j: (j, 0)),
        ],
        out_specs=pl.BlockSpec((1, SB, _BC), lambda i, j: (j, i, 0)),
        out_shape=jax.ShapeDtypeStruct((_NBC, S, _BC), jnp.float32),
    )(sign_features, heatmap)


def _edge_call(sim4, didx3, sidx3, w2):
    info = plsc.get_sparse_core_info()
    NC, NS = info.num_cores, info.num_subcores
    NW = NC * NS
    assert NW == _NEC * _NBC
    _, NB, G = didx3.shape
    EC = NB * G

    @functools.partial(
        pl.kernel,
        out_type=jax.ShapeDtypeStruct((_NBC, _NEC, 2, _DH, _BC), jnp.float32),
        mesh=plsc.VectorSubcoreMesh(core_axis_name="c", subcore_axis_name="s"),
        compiler_params=pltpu.CompilerParams(needs_layout_passes=False),
        scratch_types=[
            pltpu.VMEM((NB, G), jnp.int32),
            pltpu.VMEM((NB, G), jnp.int32),
            pltpu.VMEM((EC,), jnp.float32),
            pltpu.VMEM((G, _BC), jnp.float32),
            pltpu.VMEM((G, _BC), jnp.float32),
            pltpu.VMEM((_DH + 8, _BC), jnp.float32),
            pltpu.SemaphoreType.DMA,
            pltpu.SemaphoreType.DMA,
        ],
    )
    def k(sim_hbm, didx_hbm, sidx_hbm, w_hbm, out_hbm,
          sidx_v, didx_v, w_v, rows0, rows1, acc_v, sem0, sem1):
        c = lax.axis_index("c")
        s = lax.axis_index("s")
        wid = s * NC + c
        ec = lax.rem(wid, _NEC)
        bc = wid // _NEC
        pltpu.sync_copy(sidx_hbm.at[ec], sidx_v)
        pltpu.sync_copy(didx_hbm.at[ec], didx_v)
        pltpu.sync_copy(w_hbm.at[ec], w_v)
        zero16 = jnp.zeros((16,), jnp.float32)
        dnums = lax.GatherDimensionNumbers(
            offset_dims=(), collapsed_slice_dims=(0,), start_index_map=(0,))

        def gather_rows(b, rows, sem):
            return pltpu.async_copy(sim_hbm.at[bc].at[sidx_v.at[b]], rows,
                                    sem)

        def pass_body(p, carry0):
            d_lo = p * _DH

            def zrow(r, carry):
                for jj in range(_BC // 16):
                    acc_v[r, pl.ds(jj * 16, 16)] = zero16
                return carry

            lax.fori_loop(0, _DH + 8, zrow, 0)
            gather_rows(0, rows0, sem0)

            def pair_body(h, carry):
                b0 = 2 * h
                gather_rows(b0 + 1, rows1, sem1)
                pltpu.make_async_copy(
                    sim_hbm.at[bc].at[sidx_v.at[b0]], rows0, sem0).wait()
                process_pass(b0, rows0, d_lo)

                @pl.when(h < NB // 2 - 1)
                def _():
                    gather_rows(b0 + 2, rows0, sem0)

                pltpu.make_async_copy(
                    sim_hbm.at[bc].at[sidx_v.at[b0 + 1]], rows1, sem1).wait()
                process_pass(b0 + 1, rows1, d_lo)
                return carry

            lax.fori_loop(0, NB // 2, pair_body, 0)
            pltpu.sync_copy(acc_v.at[pl.ds(0, _DH)],
                            out_hbm.at[bc].at[ec].at[p])
            return carry0

        def process_pass(b, rows, d_lo):
            nch = _BC // 16

            def group_body(gidx, c2):
                d16 = didx_v[b, pl.ds(gidx * 16, 16)] - d_lo
                w16 = w_v[pl.ds(b * G + gidx * 16, 16)]
                dl = [d16[l] for l in range(16)]
                dd = [jnp.where(jnp.logical_and(dl[l] >= 0, dl[l] < _DH),
                                dl[l], _DH + (l % 8)) for l in range(16)]
                wspl = [
                    lax.gather(w16, jnp.full((16, 1), l, jnp.int32),
                               dnums, (1,),
                               mode=lax.GatherScatterMode.PROMISE_IN_BOUNDS)
                    for l in range(16)
                ]

                def lane_prods(l):
                    e = gidx * 16 + l
                    return [rows[e, pl.ds(jj * 16, 16)] * wspl[l]
                            for jj in range(nch)]

                prods = lane_prods(0)
                for l in range(16):
                    nxt = lane_prods(l + 1) if l < 15 else None
                    for jj in range(nch):
                        plsc.addupdate(acc_v.at[dd[l], pl.ds(jj * 16, 16)],
                                       prods[jj])
                    prods = nxt
                return c2

            lax.fori_loop(0, G // 16, group_body, 0)

        lax.fori_loop(0, 2, pass_body, 0)

    return k(sim4, didx3, sidx3, w2)


def _fin_body(p_ref, g_ref, out_ref):
    acc = jnp.sum(p_ref[...][0, :, 0], axis=0)
    g = g_ref[...]
    gn = jnp.sqrt(jnp.sum(g * g, axis=1))
    row = lax.broadcasted_iota(jnp.int32, (_BC, _BC), 0)
    col = lax.broadcasted_iota(jnp.int32, (_BC, _BC), 1)
    m = jnp.where(col == row, (_ETA * gn)[:, None], 0.0)
    out_ref[...] = lax.dot_general(m, acc, (((1,), (1,)), ((), ())),
                                   preferred_element_type=jnp.float32)


def _fin_call(partials, grad):
    B, F = grad.shape
    return pl.pallas_call(
        _fin_body,
        grid=(_NBC, 2),
        in_specs=[
            pl.BlockSpec((1, _NEC, 1, _DH, _BC), lambda j, p: (j, 0, p, 0, 0)),
            pl.BlockSpec((_BC, F), lambda j, p: (j, 0)),
        ],
        out_specs=pl.BlockSpec((_BC, _DH), lambda j, p: (j, p)),
        out_shape=jax.ShapeDtypeStruct((B, _D_PAD), jnp.float32),
    )(partials, grad)


def kernel(heatmap_features_batch, grad_output_batch, sign_features,
           disease_idx, sign_idx, edge_weight, num_diseases):
    B, F = heatmap_features_batch.shape
    S = sign_features.shape[0]
    E = disease_idx.shape[0]
    G = 128
    NB = -(-E // (_NEC * G))
    NB += NB % 2
    E_pad = _NEC * NB * G
    pad = E_pad - E

    sim4 = _sim_call(sign_features, heatmap_features_batch)

    didx_p = jnp.concatenate([disease_idx, jnp.zeros((pad,), jnp.int32)])
    sidx_p = jnp.concatenate([sign_idx, jnp.zeros((pad,), jnp.int32)])
    w_p = jnp.concatenate([edge_weight, jnp.zeros((pad,), jnp.float32)])
    didx3 = didx_p.reshape(_NEC, NB, G)
    sidx3 = sidx_p.reshape(_NEC, NB, G)
    w2 = w_p.reshape(_NEC, NB * G)
    partials = _edge_call(sim4, didx3, sidx3, w2)

    return _fin_call(partials, grad_output_batch)[:, :_D_OUT]


# speedup vs baseline: 2.5571x; 1.1350x over previous
import functools

import jax
import jax.numpy as jnp
from jax import lax
from jax.experimental import pallas as pl
from jax.experimental.pallas import tpu as pltpu
from jax.experimental.pallas import tpu_sc as plsc

_ETA = 0.01
_EPS = 1e-12
_D_OUT = 1000
_D_PAD = 1024
_DH = 512
_NEC = 16
_NBC = 2
_BC = 128


def _sim_body(sn_ref, x_ref, out_ref):
    x = x_ref[...]
    xn_blk = x / (jnp.sqrt(jnp.sum(x * x, axis=1, keepdims=True)) + _EPS)
    s = sn_ref[...]
    sn = s / (jnp.sqrt(jnp.sum(s * s, axis=1, keepdims=True)) + _EPS)
    cos = lax.dot_general(sn, xn_blk, (((1,), (1,)), ((), ())),
                          preferred_element_type=jnp.float32)
    out_ref[...] = ((cos + 1.0) * 0.5)[None]


def _sim_call(sign_features, heatmap):
    S, F = sign_features.shape
    B = heatmap.shape[0]
    SB = 1000
    return pl.pallas_call(
        _sim_body,
        grid=(S // SB, _NBC),
        in_specs=[
            pl.BlockSpec((SB, F), lambda i, j: (i, 0)),
            pl.BlockSpec((_BC, F), lambda i, j: (j, 0)),
        ],
        out_specs=pl.BlockSpec((1, SB, _BC), lambda i, j: (j, i, 0)),
        out_shape=jax.ShapeDtypeStruct((_NBC, S, _BC), jnp.float32),
    )(sign_features, heatmap)


def _edge_call(sim4, didx3, sidx3, w2):
    info = plsc.get_sparse_core_info()
    NC, NS = info.num_cores, info.num_subcores
    NW = NC * NS
    assert NW == _NEC * _NBC
    _, NB, G = didx3.shape
    EC = NB * G

    @functools.partial(
        pl.kernel,
        out_type=jax.ShapeDtypeStruct((_NBC, _NEC, 2, _DH, _BC), jnp.float32),
        mesh=plsc.VectorSubcoreMesh(core_axis_name="c", subcore_axis_name="s"),
        compiler_params=pltpu.CompilerParams(needs_layout_passes=False),
        scratch_types=[
            pltpu.VMEM((NB, G), jnp.int32),
            pltpu.VMEM((NB, G), jnp.int32),
            pltpu.VMEM((EC,), jnp.float32),
            pltpu.VMEM((G, _BC), jnp.float32),
            pltpu.VMEM((G, _BC), jnp.float32),
            pltpu.VMEM((_DH + 8, _BC), jnp.float32),
            pltpu.SemaphoreType.DMA,
            pltpu.SemaphoreType.DMA,
        ],
    )
    def k(sim_hbm, didx_hbm, sidx_hbm, w_hbm, out_hbm,
          sidx_v, didx_v, w_v, rows0, rows1, acc_v, sem0, sem1):
        c = lax.axis_index("c")
        s = lax.axis_index("s")
        wid = c * NS + s
        ec = lax.rem(wid, _NEC)
        bc = wid // _NEC
        pltpu.sync_copy(sidx_hbm.at[ec], sidx_v)
        pltpu.sync_copy(didx_hbm.at[ec], didx_v)
        pltpu.sync_copy(w_hbm.at[ec], w_v)
        zero16 = jnp.zeros((16,), jnp.float32)
        dnums = lax.GatherDimensionNumbers(
            offset_dims=(), collapsed_slice_dims=(0,), start_index_map=(0,))

        def gather_rows(b, rows, sem):
            return pltpu.async_copy(sim_hbm.at[bc].at[sidx_v.at[b]], rows,
                                    sem)

        def pass_body(p, carry0):
            d_lo = p * _DH

            def zrow(r, carry):
                for jj in range(_BC // 16):
                    acc_v[r, pl.ds(jj * 16, 16)] = zero16
                return carry

            lax.fori_loop(0, _DH + 8, zrow, 0)
            gather_rows(0, rows0, sem0)

            def pair_body(h, carry):
                b0 = 2 * h
                gather_rows(b0 + 1, rows1, sem1)
                pltpu.make_async_copy(
                    sim_hbm.at[bc].at[sidx_v.at[b0]], rows0, sem0).wait()
                process_pass(b0, rows0, d_lo)

                @pl.when(h < NB // 2 - 1)
                def _():
                    gather_rows(b0 + 2, rows0, sem0)

                pltpu.make_async_copy(
                    sim_hbm.at[bc].at[sidx_v.at[b0 + 1]], rows1, sem1).wait()
                process_pass(b0 + 1, rows1, d_lo)
                return carry

            lax.fori_loop(0, NB // 2, pair_body, 0)
            pltpu.sync_copy(acc_v.at[pl.ds(0, _DH)],
                            out_hbm.at[bc].at[ec].at[p])
            return carry0

        def process_pass(b, rows, d_lo):
            nch = _BC // 16

            def group_body(gidx, c2):
                d16 = didx_v[b, pl.ds(gidx * 16, 16)] - d_lo
                w16 = w_v[pl.ds(b * G + gidx * 16, 16)]
                dl = [d16[l] for l in range(16)]
                dd = [jnp.where(jnp.logical_and(dl[l] >= 0, dl[l] < _DH),
                                dl[l], _DH + (l % 8)) for l in range(16)]
                wspl = [
                    lax.gather(w16, jnp.full((16, 1), l, jnp.int32),
                               dnums, (1,),
                               mode=lax.GatherScatterMode.PROMISE_IN_BOUNDS)
                    for l in range(16)
                ]

                def lane_prods(l):
                    e = gidx * 16 + l
                    return [rows[e, pl.ds(jj * 16, 16)] * wspl[l]
                            for jj in range(nch)]

                prods = lane_prods(0)
                for l in range(16):
                    nxt = lane_prods(l + 1) if l < 15 else None
                    for jj in range(nch):
                        plsc.addupdate(acc_v.at[dd[l], pl.ds(jj * 16, 16)],
                                       prods[jj])
                    prods = nxt
                return c2

            lax.fori_loop(0, G // 16, group_body, 0)

        lax.fori_loop(0, 2, pass_body, 0)

    return k(sim4, didx3, sidx3, w2)


def _fin_body(p_ref, g_ref, out_ref):
    acc = jnp.sum(p_ref[...][0, :, 0], axis=0)
    g = g_ref[...]
    gn = jnp.sqrt(jnp.sum(g * g, axis=1))
    row = lax.broadcasted_iota(jnp.int32, (_BC, _BC), 0)
    col = lax.broadcasted_iota(jnp.int32, (_BC, _BC), 1)
    m = jnp.where(col == row, (_ETA * gn)[:, None], 0.0)
    out_ref[...] = lax.dot_general(m, acc, (((1,), (1,)), ((), ())),
                                   preferred_element_type=jnp.float32)


def _fin_call(partials, grad):
    B, F = grad.shape
    return pl.pallas_call(
        _fin_body,
        grid=(_NBC, 2),
        in_specs=[
            pl.BlockSpec((1, _NEC, 1, _DH, _BC), lambda j, p: (j, 0, p, 0, 0)),
            pl.BlockSpec((_BC, F), lambda j, p: (j, 0)),
        ],
        out_specs=pl.BlockSpec((_BC, _DH), lambda j, p: (j, p)),
        out_shape=jax.ShapeDtypeStruct((B, _D_PAD), jnp.float32),
    )(partials, grad)


def kernel(heatmap_features_batch, grad_output_batch, sign_features,
           disease_idx, sign_idx, edge_weight, num_diseases):
    B, F = heatmap_features_batch.shape
    S = sign_features.shape[0]
    E = disease_idx.shape[0]
    G = 128
    NB = -(-E // (_NEC * G))
    NB += NB % 2
    E_pad = _NEC * NB * G
    pad = E_pad - E

    sim4 = _sim_call(sign_features, heatmap_features_batch)

    didx_p = jnp.concatenate([disease_idx, jnp.zeros((pad,), jnp.int32)])
    sidx_p = jnp.concatenate([sign_idx, jnp.zeros((pad,), jnp.int32)])
    w_p = jnp.concatenate([edge_weight, jnp.zeros((pad,), jnp.float32)])
    didx3 = didx_p.reshape(_NEC, NB, G)
    sidx3 = sidx_p.reshape(_NEC, NB, G)
    w2 = w_p.reshape(_NEC, NB * G)
    partials = _edge_call(sim4, didx3, sidx3, w2)

    return _fin_call(partials, grad_output_batch)[:, :_D_OUT]
